# baseline (speedup 1.0000x reference)
import numpy as np
import jax
import jax.numpy as jnp
from jax.experimental import pallas as pl
from jax.experimental.pallas import tpu as pltpu


def _lora_body(xids_ref, wids_ref, rows_ref, x_ref, a_ref, b_ref, out_ref):
    i = pl.program_id(0)

    @pl.when(i == 0)
    def _init():
        out_ref[...] = jnp.zeros_like(out_ref)

    a = a_ref[0]
    b = b_ref[0]
    v = jax.lax.dot_general(x_ref[0], a, (((1,), (0,)), ((), ())),
                            preferred_element_type=jnp.float32)
    y = jax.lax.dot_general(v, b, (((1,), (0,)), ((), ())),
                            preferred_element_type=jnp.float32)
    row = rows_ref[i]
    out_ref[pl.ds(row, 1), :] += 2.0 * y


def kernel(x, xids, wids, lora_A, lora_B):
    batch, _, d_model = x.shape
    lora_batch = xids.shape[0]
    r = lora_A.shape[2]

    r_mult = 4
    batch_large = (lora_batch - batch) // (r_mult - 1)
    n_summed = batch_large * r_mult
    ar = np.arange(lora_batch)
    rows = jnp.asarray(
        np.where(ar < n_summed, ar // r_mult, ar - n_summed + batch_large)
        .astype(np.int32))

    grid_spec = pltpu.PrefetchScalarGridSpec(
        num_scalar_prefetch=3,
        grid=(lora_batch,),
        in_specs=[
            pl.BlockSpec((1, 1, d_model), lambda i, xids, wids, rows: (xids[i], 0, 0)),
            pl.BlockSpec((1, d_model, r), lambda i, xids, wids, rows: (wids[i], 0, 0)),
            pl.BlockSpec((1, r, d_model), lambda i, xids, wids, rows: (wids[i], 0, 0)),
        ],
        out_specs=pl.BlockSpec((batch, d_model), lambda i, xids, wids, rows: (0, 0)),
    )
    out = pl.pallas_call(
        _lora_body,
        grid_spec=grid_spec,
        out_shape=jax.ShapeDtypeStruct((batch, d_model), jnp.float32),
        compiler_params=pltpu.CompilerParams(
            dimension_semantics=("arbitrary",),
        ),
    )(xids, wids, rows, x, lora_A, lora_B)
    return out.reshape(batch, 1, d_model)


# speedup vs baseline: 1.5063x; 1.5063x over previous
import numpy as np
import jax
import jax.numpy as jnp
from jax.experimental import pallas as pl
from jax.experimental.pallas import tpu as pltpu


def _lora_body(xids_ref, wids_ref, rows_ref, xs_ref, a_ref, b_ref,
               sel_ref, s_ref, out_ref):
    i = pl.program_id(0)

    @pl.when(i == 0)
    def _init():
        out_ref[...] = jnp.zeros_like(out_ref)

    xs = xs_ref[0]
    a8 = a_ref[0]
    b = b_ref[0]
    w = jax.lax.dot_general(xs, a8, (((1,), (0,)), ((), ())),
                            preferred_element_type=jnp.float32)
    vf = jnp.sum(w * sel_ref[...], axis=0, keepdims=True)
    v = jax.lax.dot_general(vf, s_ref[...], (((1,), (0,)), ((), ())),
                            preferred_element_type=jnp.float32)
    y = jax.lax.dot_general(v, b, (((1,), (0,)), ((), ())),
                            preferred_element_type=jnp.float32)
    row = rows_ref[i]
    out_ref[pl.ds(row, 1), :] += 2.0 * y


def kernel(x, xids, wids, lora_A, lora_B):
    batch, _, d_model = x.shape
    lora_batch = xids.shape[0]
    num_ad = lora_A.shape[0]
    r = lora_A.shape[2]
    sub = 128 // r
    p_rows = d_model // sub

    r_mult = 4
    batch_large = (lora_batch - batch) // (r_mult - 1)
    n_summed = batch_large * r_mult
    ar = np.arange(lora_batch)
    rows = jnp.asarray(
        np.where(ar < n_summed, ar // r_mult, ar - n_summed + batch_large)
        .astype(np.int32))

    xs = x.reshape(batch, p_rows, sub).swapaxes(1, 2)
    a8 = lora_A.reshape(num_ad, p_rows, sub * r)

    q = np.arange(sub * r)
    sel = jnp.asarray((q[None, :] // r == np.arange(sub)[:, None])
                      .astype(np.float32))
    s = jnp.asarray((q[:, None] % r == np.arange(r)[None, :])
                    .astype(np.float32))

    grid_spec = pltpu.PrefetchScalarGridSpec(
        num_scalar_prefetch=3,
        grid=(lora_batch,),
        in_specs=[
            pl.BlockSpec((1, sub, p_rows),
                         lambda i, xids, wids, rows: (xids[i], 0, 0)),
            pl.BlockSpec((1, p_rows, sub * r),
                         lambda i, xids, wids, rows: (wids[i], 0, 0)),
            pl.BlockSpec((1, r, d_model),
                         lambda i, xids, wids, rows: (wids[i], 0, 0)),
            pl.BlockSpec((sub, sub * r),
                         lambda i, xids, wids, rows: (0, 0)),
            pl.BlockSpec((sub * r, r),
                         lambda i, xids, wids, rows: (0, 0)),
        ],
        out_specs=pl.BlockSpec((batch, d_model),
                               lambda i, xids, wids, rows: (0, 0)),
    )
    out = pl.pallas_call(
        _lora_body,
        grid_spec=grid_spec,
        out_shape=jax.ShapeDtypeStruct((batch, d_model), jnp.float32),
        compiler_params=pltpu.CompilerParams(
            dimension_semantics=("arbitrary",),
        ),
    )(xids, wids, rows, xs, a8, lora_B, sel, s)
    return out.reshape(batch, 1, d_model)


# speedup vs baseline: 2.2519x; 1.4951x over previous
import functools
import numpy as np
import jax
import jax.numpy as jnp
from jax.experimental import pallas as pl
from jax.experimental.pallas import tpu as pltpu

_G = 8


def _lora_body(n_group_steps, batch_large, xids_ref, wids_ref, *refs):
    xs_refs = refs[0:_G]
    a_refs = refs[_G:2 * _G]
    b_refs = refs[2 * _G:3 * _G]
    sel_ref, s_ref, out_ref = refs[3 * _G], refs[3 * _G + 1], refs[3 * _G + 2]
    i = pl.program_id(0)

    sel = sel_ref[...]
    s = s_ref[...]
    ys = []
    for k in range(_G):
        xs = xs_refs[k][0]
        a8 = a_refs[k][0]
        b = b_refs[k][0]
        w = jax.lax.dot_general(xs, a8, (((1,), (0,)), ((), ())),
                                preferred_element_type=jnp.float32)
        vf = jnp.sum(w * sel, axis=0, keepdims=True)
        v = jax.lax.dot_general(vf, s, (((1,), (0,)), ((), ())),
                                preferred_element_type=jnp.float32)
        y = jax.lax.dot_general(v, b, (((1,), (0,)), ((), ())),
                                preferred_element_type=jnp.float32)
        ys.append(y)

    @pl.when(i < n_group_steps)
    def _groups():
        row = 2 * i
        out_ref[pl.ds(row, 1), :] = (ys[0] + ys[1]) + (ys[2] + ys[3])
        out_ref[pl.ds(row + 1, 1), :] = (ys[4] + ys[5]) + (ys[6] + ys[7])

    @pl.when(i >= n_group_steps)
    def _passthrough():
        base = batch_large + (i - n_group_steps) * _G
        for k in range(_G):
            out_ref[pl.ds(base + k, 1), :] = ys[k]


def kernel(x, xids, wids, lora_A, lora_B):
    batch, _, d_model = x.shape
    lora_batch = xids.shape[0]
    num_ad = lora_A.shape[0]
    r = lora_A.shape[2]
    sub = 128 // r
    p_rows = d_model // sub

    r_mult = 4
    batch_large = (lora_batch - batch) // (r_mult - 1)
    n_summed = batch_large * r_mult
    n_steps = lora_batch // _G
    n_group_steps = n_summed // _G

    xs = x.reshape(batch, p_rows, sub).swapaxes(1, 2)
    a8 = lora_A.reshape(num_ad, p_rows, sub * r)

    q = np.arange(sub * r)
    sel = jnp.asarray(2.0 * (q[None, :] // r == np.arange(sub)[:, None])
                      .astype(np.float32))
    s = jnp.asarray((q[:, None] % r == np.arange(r)[None, :])
                    .astype(np.float32))

    def xs_spec(k):
        return pl.BlockSpec(
            (1, sub, p_rows),
            lambda i, xids, wids, k=k: (xids[_G * i + k], 0, 0))

    def a_spec(k):
        return pl.BlockSpec(
            (1, p_rows, sub * r),
            lambda i, xids, wids, k=k: (wids[_G * i + k], 0, 0))

    def b_spec(k):
        return pl.BlockSpec(
            (1, r, d_model),
            lambda i, xids, wids, k=k: (wids[_G * i + k], 0, 0))

    grid_spec = pltpu.PrefetchScalarGridSpec(
        num_scalar_prefetch=2,
        grid=(n_steps,),
        in_specs=(
            [xs_spec(k) for k in range(_G)]
            + [a_spec(k) for k in range(_G)]
            + [b_spec(k) for k in range(_G)]
            + [pl.BlockSpec((sub, sub * r), lambda i, xids, wids: (0, 0)),
               pl.BlockSpec((sub * r, r), lambda i, xids, wids: (0, 0))]
        ),
        out_specs=pl.BlockSpec((batch, d_model),
                               lambda i, xids, wids: (0, 0)),
    )
    out = pl.pallas_call(
        functools.partial(_lora_body, n_group_steps, batch_large),
        grid_spec=grid_spec,
        out_shape=jax.ShapeDtypeStruct((batch, d_model), jnp.float32),
        compiler_params=pltpu.CompilerParams(
            dimension_semantics=("arbitrary",),
        ),
    )(xids, wids,
      *([xs] * _G), *([a8] * _G), *([lora_B] * _G), sel, s)
    return out.reshape(batch, 1, d_model)


# speedup vs baseline: 2.3957x; 1.0639x over previous
import functools
import numpy as np
import jax
import jax.numpy as jnp
from jax.experimental import pallas as pl
from jax.experimental.pallas import tpu as pltpu

_G = 8


def _lora_body(n_group_steps, batch_large, xids_ref, wids_ref, *refs):
    xs_refs = refs[0:_G]
    a_refs = refs[_G:2 * _G]
    b_refs = refs[2 * _G:3 * _G]
    sel_ref, out_ref = refs[3 * _G], refs[3 * _G + 1]
    i = pl.program_id(0)

    sel = sel_ref[...]
    r = b_refs[0].shape[1]
    ys = []
    for k in range(_G):
        xs = xs_refs[k][0]
        a8 = a_refs[k][0]
        b = b_refs[k][0]
        w = jax.lax.dot_general(xs, a8, (((1,), (0,)), ((), ())),
                                preferred_element_type=jnp.float32)
        vf = jnp.sum(w * sel, axis=0, keepdims=True)
        v = vf[:, 0 * r:1 * r]
        for j in range(1, 128 // r):
            v = v + vf[:, j * r:(j + 1) * r]
        y = jax.lax.dot_general(v, b, (((1,), (0,)), ((), ())),
                                preferred_element_type=jnp.float32)
        ys.append(y)

    @pl.when(i < n_group_steps)
    def _groups():
        row = 2 * i
        out_ref[pl.ds(row, 1), :] = (ys[0] + ys[1]) + (ys[2] + ys[3])
        out_ref[pl.ds(row + 1, 1), :] = (ys[4] + ys[5]) + (ys[6] + ys[7])

    @pl.when(i >= n_group_steps)
    def _passthrough():
        base = batch_large + (i - n_group_steps) * _G
        for k in range(_G):
            out_ref[pl.ds(base + k, 1), :] = ys[k]


def kernel(x, xids, wids, lora_A, lora_B):
    batch, _, d_model = x.shape
    lora_batch = xids.shape[0]
    num_ad = lora_A.shape[0]
    r = lora_A.shape[2]
    sub = 128 // r
    p_rows = d_model // sub

    r_mult = 4
    batch_large = (lora_batch - batch) // (r_mult - 1)
    n_summed = batch_large * r_mult
    n_steps = lora_batch // _G
    n_group_steps = n_summed // _G

    xs = x.reshape(batch, p_rows, sub).swapaxes(1, 2)
    a8 = lora_A.reshape(num_ad, p_rows, sub * r)

    q = np.arange(sub * r)
    sel = jnp.asarray(2.0 * (q[None, :] // r == np.arange(sub)[:, None])
                      .astype(np.float32))

    def xs_spec(k):
        return pl.BlockSpec(
            (1, sub, p_rows),
            lambda i, xids, wids, k=k: (xids[_G * i + k], 0, 0))

    def a_spec(k):
        return pl.BlockSpec(
            (1, p_rows, sub * r),
            lambda i, xids, wids, k=k: (wids[_G * i + k], 0, 0))

    def b_spec(k):
        return pl.BlockSpec(
            (1, r, d_model),
            lambda i, xids, wids, k=k: (wids[_G * i + k], 0, 0))

    grid_spec = pltpu.PrefetchScalarGridSpec(
        num_scalar_prefetch=2,
        grid=(n_steps,),
        in_specs=(
            [xs_spec(k) for k in range(_G)]
            + [a_spec(k) for k in range(_G)]
            + [b_spec(k) for k in range(_G)]
            + [pl.BlockSpec((sub, sub * r), lambda i, xids, wids: (0, 0))]
        ),
        out_specs=pl.BlockSpec((batch, d_model),
                               lambda i, xids, wids: (0, 0)),
    )
    out = pl.pallas_call(
        functools.partial(_lora_body, n_group_steps, batch_large),
        grid_spec=grid_spec,
        out_shape=jax.ShapeDtypeStruct((batch, d_model), jnp.float32),
        compiler_params=pltpu.CompilerParams(
            dimension_semantics=("arbitrary",),
        ),
    )(xids, wids,
      *([xs] * _G), *([a8] * _G), *([lora_B] * _G), sel)
    return out.reshape(batch, 1, d_model)


# speedup vs baseline: 8.0630x; 3.3656x over previous
import functools
import numpy as np
import jax
import jax.numpy as jnp
from jax.experimental import pallas as pl
from jax.experimental.pallas import tpu as pltpu

_G = 8


def _lora_body(n_group_steps, batch_large, xids_ref, wids_ref, *refs):
    x_refs = refs[0:_G]
    a_refs = refs[_G:2 * _G]
    b_refs = refs[2 * _G:3 * _G]
    out_ref = refs[3 * _G]
    i = pl.program_id(0)

    ys = []
    for k in range(_G):
        xr = x_refs[k][0]
        at = a_refs[k][0]
        b = b_refs[k][0]
        t = at * xr
        v = jnp.sum(t, axis=1, keepdims=True) * 2.0
        y = jax.lax.dot_general(v, b, (((0,), (0,)), ((), ())),
                                preferred_element_type=jnp.float32)
        ys.append(y)

    @pl.when(i < n_group_steps)
    def _groups():
        row = 2 * i
        out_ref[pl.ds(row, 1), :] = (ys[0] + ys[1]) + (ys[2] + ys[3])
        out_ref[pl.ds(row + 1, 1), :] = (ys[4] + ys[5]) + (ys[6] + ys[7])

    @pl.when(i >= n_group_steps)
    def _passthrough():
        base = batch_large + (i - n_group_steps) * _G
        for k in range(_G):
            out_ref[pl.ds(base + k, 1), :] = ys[k]


def kernel(x, xids, wids, lora_A, lora_B):
    batch, _, d_model = x.shape
    lora_batch = xids.shape[0]
    r = lora_A.shape[2]

    r_mult = 4
    batch_large = (lora_batch - batch) // (r_mult - 1)
    n_summed = batch_large * r_mult
    n_steps = lora_batch // _G
    n_group_steps = n_summed // _G

    at = lora_A.transpose(0, 2, 1)

    def x_spec(k):
        return pl.BlockSpec(
            (1, 1, d_model),
            lambda i, xids, wids, k=k: (xids[_G * i + k], 0, 0))

    def ab_spec(k):
        return pl.BlockSpec(
            (1, r, d_model),
            lambda i, xids, wids, k=k: (wids[_G * i + k], 0, 0))

    grid_spec = pltpu.PrefetchScalarGridSpec(
        num_scalar_prefetch=2,
        grid=(n_steps,),
        in_specs=(
            [x_spec(k) for k in range(_G)]
            + [ab_spec(k) for k in range(_G)]
            + [ab_spec(k) for k in range(_G)]
        ),
        out_specs=pl.BlockSpec((batch, d_model),
                               lambda i, xids, wids: (0, 0)),
    )
    out = pl.pallas_call(
        functools.partial(_lora_body, n_group_steps, batch_large),
        grid_spec=grid_spec,
        out_shape=jax.ShapeDtypeStruct((batch, d_model), jnp.float32),
        compiler_params=pltpu.CompilerParams(
            dimension_semantics=("arbitrary",),
        ),
    )(xids, wids,
      *([x] * _G), *([at] * _G), *([lora_B] * _G))
    return out.reshape(batch, 1, d_model)
